# Initial kernel scaffold; baseline (speedup 1.0000x reference)
import jax
import jax.numpy as jnp
from jax.experimental import pallas as pl


def kernel(hidden_states, W):
    raise NotImplementedError("write your pallas kernel here")


# speedup vs baseline: 1.8741x; 1.8741x over previous
import jax
import jax.numpy as jnp
from jax.experimental import pallas as pl

_NUM_EXPERTS = 64
_TOP_K = 2
_BLOCK_T = 1024


def _router_body(x_ref, w_ref, logits_ref, aff_ref, i0_ref, i1_ref):
    x = x_ref[...]
    w = w_ref[...]
    logits = jnp.dot(x, w, preferred_element_type=jnp.float32)
    logits_ref[...] = logits

    m0 = jnp.max(logits, axis=1, keepdims=True)
    e = jnp.exp(logits - m0)
    s = jnp.sum(e, axis=1, keepdims=True)
    aff_ref[...] = e / s

    iota = jax.lax.broadcasted_iota(jnp.int32, logits.shape, 1)
    i0 = jnp.min(jnp.where(logits == m0, iota, _NUM_EXPERTS), axis=1)
    masked = jnp.where(iota == i0[:, None], -jnp.inf, logits)
    m1 = jnp.max(masked, axis=1, keepdims=True)
    i1 = jnp.min(jnp.where(masked == m1, iota, _NUM_EXPERTS), axis=1)
    i0_ref[...] = i0
    i1_ref[...] = i1


def kernel(hidden_states, W):
    Bq, Sq, D = hidden_states.shape
    T = Bq * Sq
    x = hidden_states.reshape(T, D)
    E = W.shape[1]

    grid = (T // _BLOCK_T,)
    logits, aff, i0, i1 = pl.pallas_call(
        _router_body,
        grid=grid,
        in_specs=[
            pl.BlockSpec((_BLOCK_T, D), lambda i: (i, 0)),
            pl.BlockSpec((D, E), lambda i: (0, 0)),
        ],
        out_specs=[
            pl.BlockSpec((_BLOCK_T, E), lambda i: (i, 0)),
            pl.BlockSpec((_BLOCK_T, E), lambda i: (i, 0)),
            pl.BlockSpec((_BLOCK_T,), lambda i: (i,)),
            pl.BlockSpec((_BLOCK_T,), lambda i: (i,)),
        ],
        out_shape=[
            jax.ShapeDtypeStruct((T, E), jnp.float32),
            jax.ShapeDtypeStruct((T, E), jnp.float32),
            jax.ShapeDtypeStruct((T,), jnp.int32),
            jax.ShapeDtypeStruct((T,), jnp.int32),
        ],
    )(x, W)

    expert_index = jnp.stack([i0, i1], axis=-1)
    return logits, aff, expert_index
